# Initial kernel scaffold; baseline (speedup 1.0000x reference)
import jax
import jax.numpy as jnp
from jax.experimental import pallas as pl


def kernel(x, edge_index, W, b, W_ih, W_hh, b_ih, b_hh, h_mem):
    raise NotImplementedError("write your pallas kernel here")


# speedup vs baseline: 9.6119x; 9.6119x over previous
import functools

import jax
import jax.numpy as jnp
from jax import lax
from jax.experimental import pallas as pl
from jax.experimental.pallas import tpu as pltpu
from jax.experimental.pallas import tpu_sc as plsc

N = 10000
E = 320000
D_IN = 128
H = 64

NC = 2
NS = 16
NW = NC * NS

NPAD = 10240
ND8 = NPAD // 8
NH2 = NPAD // 2
D_ROWS_TILE = ND8 // NS
A_ROWS_TILE = NH2 // NS
E_TILE = E // NW
CK = 128

_MESH = plsc.VectorSubcoreMesh(
    core_axis_name="c", subcore_axis_name="s", num_cores=NC, num_subcores=NS
)


def _sc_deg_body(dst_hbm, ones8_hbm, deg_hbm, dst_v, orow_v, st_v,
                 deg_sp, ones_sp, sem):
    cid = lax.axis_index("c")
    sid = lax.axis_index("s")
    g = sid * NC + cid
    zero16 = jnp.zeros((16,), jnp.float32)

    def zrow(i, carry):
        for k in range(8):
            st_v[i, pl.ds(k * 16, 16)] = zero16
        return carry

    lax.fori_loop(0, D_ROWS_TILE, zrow, 0)
    pltpu.sync_copy(st_v, deg_sp.at[pl.ds(sid * D_ROWS_TILE, D_ROWS_TILE)])

    @pl.when(sid == 0)
    def _():
        pltpu.sync_copy(ones8_hbm, st_v.at[pl.ds(0, 8)])
        pltpu.sync_copy(st_v.at[pl.ds(0, 8)], ones_sp)

    pltpu.sync_copy(dst_hbm.at[pl.ds(g * E_TILE, E_TILE)], dst_v)
    plsc.subcore_barrier()

    seven = jnp.full((16,), 7, jnp.int32)
    three = jnp.full((16,), 3, jnp.int32)

    def dbody(j, carry):
        d16 = dst_v[pl.ds(j * 16, 16)]
        pltpu.async_copy(ones_sp.at[d16 & seven], orow_v, sem).wait()
        pltpu.sync_copy(orow_v, deg_sp.at[lax.shift_right_logical(d16, three)],
                        add=True)
        return carry

    lax.fori_loop(0, E_TILE // 16, dbody, 0)
    plsc.subcore_barrier()
    sl = pl.ds(sid * D_ROWS_TILE, D_ROWS_TILE)
    pltpu.sync_copy(deg_sp.at[sl], st_v)
    pltpu.sync_copy(st_v, deg_hbm.at[cid, sl])


_sc_deg = pl.kernel(
    _sc_deg_body,
    out_type=jax.ShapeDtypeStruct((NC, ND8, 128), jnp.float32),
    mesh=_MESH,
    scratch_types=[
        pltpu.VMEM((E_TILE,), jnp.int32),
        pltpu.VMEM((16, 128), jnp.float32),
        pltpu.VMEM((D_ROWS_TILE, 128), jnp.float32),
        pltpu.VMEM_SHARED((ND8, 128), jnp.float32),
        pltpu.VMEM_SHARED((8, 128), jnp.float32),
        pltpu.SemaphoreType.DMA,
    ],
)


def _sc_agg_body(src_hbm, dst_hbm, y_hbm, acc_hbm,
                 src_v, dst_v, rows_v, ca_v, acc_sp, sem):
    cid = lax.axis_index("c")
    sid = lax.axis_index("s")
    g = sid * NC + cid
    zero16 = jnp.zeros((16,), jnp.float32)

    def zrow(i, carry):
        for k in range(8):
            ca_v[i, pl.ds(k * 16, 16)] = zero16
        return carry

    lax.fori_loop(0, CK, zrow, 0)
    a0 = sid * A_ROWS_TILE
    pltpu.sync_copy(ca_v, acc_sp.at[pl.ds(a0, CK)])
    pltpu.sync_copy(ca_v, acc_sp.at[pl.ds(a0 + CK, CK)])
    pltpu.sync_copy(ca_v.at[pl.ds(0, 64)], acc_sp.at[pl.ds(a0 + 2 * CK, 64)])
    pltpu.sync_copy(src_hbm.at[pl.ds(g * E_TILE, E_TILE)], src_v)
    pltpu.sync_copy(dst_hbm.at[pl.ds(g * E_TILE, E_TILE)], dst_v)
    plsc.subcore_barrier()

    one = jnp.full((16,), 1, jnp.int32)
    npad = jnp.full((16,), NPAD, jnp.int32)

    def abody(j, carry):
        s16 = src_v[pl.ds(j * 16, 16)]
        d16 = dst_v[pl.ds(j * 16, 16)]
        idx_g = s16 + (d16 & one) * npad
        pltpu.async_copy(y_hbm.at[idx_g], rows_v, sem).wait()
        pltpu.sync_copy(rows_v, acc_sp.at[lax.shift_right_logical(d16, one)],
                        add=True)
        return carry

    lax.fori_loop(0, E_TILE // 16, abody, 0)
    plsc.subcore_barrier()
    for c in range(2):
        r0 = a0 + c * CK
        pltpu.sync_copy(acc_sp.at[pl.ds(r0, CK)], ca_v)
        pltpu.sync_copy(ca_v, acc_hbm.at[cid, pl.ds(r0, CK)])
    r0 = a0 + 2 * CK
    pltpu.sync_copy(acc_sp.at[pl.ds(r0, 64)], ca_v.at[pl.ds(0, 64)])
    pltpu.sync_copy(ca_v.at[pl.ds(0, 64)], acc_hbm.at[cid, pl.ds(r0, 64)])


_sc_agg = pl.kernel(
    _sc_agg_body,
    out_type=jax.ShapeDtypeStruct((NC, NH2, 128), jnp.float32),
    mesh=_MESH,
    scratch_types=[
        pltpu.VMEM((E_TILE,), jnp.int32),
        pltpu.VMEM((E_TILE,), jnp.int32),
        pltpu.VMEM((16, 128), jnp.float32),
        pltpu.VMEM((CK, 128), jnp.float32),
        pltpu.VMEM_SHARED((NH2, 128), jnp.float32),
        pltpu.SemaphoreType.DMA,
    ],
)


RB = 2000
NB = N // RB


def _tc_mid_body(x_ref, w_ref, degp_ref, yd_ref, y_ref):
    half = pl.program_id(0) // NB
    xw = jnp.dot(x_ref[...], w_ref[...], preferred_element_type=jnp.float32)
    deg = degp_ref[0, :, 0:1] + degp_ref[1, :, 0:1] + 1.0
    v = xw * lax.rsqrt(deg)
    y_ref[...] = v
    z = jnp.zeros((RB, H), jnp.float32)

    @pl.when(half == 0)
    def _():
        yd_ref[...] = jnp.concatenate([v, z], axis=1)[None]

    @pl.when(half == 1)
    def _():
        yd_ref[...] = jnp.concatenate([z, v], axis=1)[None]


def _tc_mid(x, w, degp):
    return pl.pallas_call(
        _tc_mid_body,
        grid=(2 * NB,),
        in_specs=[
            pl.BlockSpec((RB, D_IN), lambda j: (j % NB, 0)),
            pl.BlockSpec((D_IN, H), lambda j: (0, 0)),
            pl.BlockSpec((NC, RB, 16), lambda j: (0, j % NB, 0)),
        ],
        out_specs=[
            pl.BlockSpec((1, RB, 128), lambda j: (j // NB, j % NB, 0)),
            pl.BlockSpec((RB, H), lambda j: (j % NB, 0)),
        ],
        out_shape=[
            jax.ShapeDtypeStruct((2, NPAD, 128), jnp.float32),
            jax.ShapeDtypeStruct((N, H), jnp.float32),
        ],
    )(x, w, degp)


def _tc_final_body(acc_ref, degp_ref, y_ref, b_ref,
                   wir_ref, wiz_ref, win_ref, whr_ref, whz_ref, whn_ref,
                   bir_ref, biz_ref, bin_ref, bhr_ref, bhz_ref, bhn_ref,
                   h_ref, out_ref):
    deg = degp_ref[0, :, 0:1] + degp_ref[1, :, 0:1] + 1.0
    dinv = lax.rsqrt(deg)
    y = y_ref[...]
    g = (acc_ref[0] + acc_ref[1] + y) * dinv + b_ref[...]
    h = h_ref[...]
    dot = functools.partial(jnp.dot, preferred_element_type=jnp.float32)
    r = jax.nn.sigmoid(dot(g, wir_ref[...]) + bir_ref[...]
                       + dot(h, whr_ref[...]) + bhr_ref[...])
    z = jax.nn.sigmoid(dot(g, wiz_ref[...]) + biz_ref[...]
                       + dot(h, whz_ref[...]) + bhz_ref[...])
    n = jnp.tanh(dot(g, win_ref[...]) + bin_ref[...]
                 + r * (dot(h, whn_ref[...]) + bhn_ref[...]))
    out_ref[...] = (1.0 - z) * n + z * h


def _tc_final(acc, degp, y, b2, wmats, bvecs, h_mem):
    full = lambda shape: pl.BlockSpec(shape, lambda i: tuple(0 for _ in shape))
    return pl.pallas_call(
        _tc_final_body,
        grid=(NB,),
        in_specs=[
            pl.BlockSpec((NC, RB, H), lambda i: (0, i, 0)),
            pl.BlockSpec((NC, RB, 16), lambda i: (0, i, 0)),
            pl.BlockSpec((RB, H), lambda i: (i, 0)),
            full((1, H)),
            *[full((H, H)) for _ in range(6)],
            *[full((1, H)) for _ in range(6)],
            pl.BlockSpec((RB, H), lambda i: (i, 0)),
        ],
        out_specs=pl.BlockSpec((RB, H), lambda i: (i, 0)),
        out_shape=jax.ShapeDtypeStruct((N, H), jnp.float32),
    )(acc, degp, y, b2, *wmats, *bvecs, h_mem)


def kernel(x, edge_index, W, b, W_ih, W_hh, b_ih, b_hh, h_mem):
    ei = edge_index.astype(jnp.int32)
    src = ei[0].reshape(E)
    dst = ei[1].reshape(E)

    ones8 = jnp.repeat(jnp.eye(8, dtype=jnp.float32), 16, axis=1)

    degp8 = _sc_deg(dst, ones8)
    degp = degp8.reshape(NC, NPAD, 16)
    yd, y = _tc_mid(x, W, degp)
    acc = _sc_agg(src, dst, yd.reshape(2 * NPAD, 128))
    acc = acc.reshape(NC, NPAD, H)

    wm = [W_ih[0:H].T, W_ih[H:2 * H].T, W_ih[2 * H:3 * H].T,
          W_hh[0:H].T, W_hh[H:2 * H].T, W_hh[2 * H:3 * H].T]
    bv = [b_ih[0:H].reshape(1, H), b_ih[H:2 * H].reshape(1, H),
          b_ih[2 * H:3 * H].reshape(1, H),
          b_hh[0:H].reshape(1, H), b_hh[H:2 * H].reshape(1, H),
          b_hh[2 * H:3 * H].reshape(1, H)]
    return _tc_final(acc, degp, y, b.reshape(1, H), wm, bv, h_mem)


# speedup vs baseline: 15.6256x; 1.6256x over previous
import functools

import jax
import jax.numpy as jnp
from jax import lax
from jax.experimental import pallas as pl
from jax.experimental.pallas import tpu as pltpu
from jax.experimental.pallas import tpu_sc as plsc

N = 10000
E = 320000
D_IN = 128
H = 64

NC = 2
NS = 16
NW = NC * NS

NPAD = 10240
ND8 = NPAD // 8
NH2 = NPAD // 2
D_ROWS_TILE = ND8 // NS
A_ROWS_TILE = NH2 // NS
E_TILE = E // NW
CK = 128

_MESH = plsc.VectorSubcoreMesh(
    core_axis_name="c", subcore_axis_name="s", num_cores=NC, num_subcores=NS
)


def _sc_deg_body(dst_hbm, ones8_hbm, deg_hbm, dst_v, orow_v, orow2_v, st_v,
                 deg_sp, ones_sp, sem, sem2):
    cid = lax.axis_index("c")
    sid = lax.axis_index("s")
    g = sid * NC + cid
    zero16 = jnp.zeros((16,), jnp.float32)

    def zrow(i, carry):
        for k in range(8):
            st_v[i, pl.ds(k * 16, 16)] = zero16
        return carry

    lax.fori_loop(0, D_ROWS_TILE, zrow, 0)
    pltpu.sync_copy(st_v, deg_sp.at[pl.ds(sid * D_ROWS_TILE, D_ROWS_TILE)])

    @pl.when(sid == 0)
    def _():
        pltpu.sync_copy(ones8_hbm, st_v.at[pl.ds(0, 8)])
        pltpu.sync_copy(st_v.at[pl.ds(0, 8)], ones_sp)

    pltpu.sync_copy(dst_hbm.at[pl.ds(g * E_TILE, E_TILE)], dst_v)
    plsc.subcore_barrier()

    seven = jnp.full((16,), 7, jnp.int32)
    three = jnp.full((16,), 3, jnp.int32)
    NG = E_TILE // 16

    def d_of(j):
        return dst_v[pl.ds(j * 16, 16)]

    def fire(j, buf, s):
        d16 = d_of(j)
        pltpu.async_copy(ones_sp.at[d16 & seven], buf, s)

    def scat(j, buf):
        pltpu.sync_copy(buf, deg_sp.at[lax.shift_right_logical(d_of(j), three)],
                        add=True)

    fire(0, orow_v, sem)

    def dbody(i, carry):
        j = 2 * i
        fire(j + 1, orow2_v, sem2)
        pltpu.make_async_copy(deg_hbm.at[cid, pl.ds(0, 16)], orow_v, sem).wait()
        scat(j, orow_v)
        fire(j + 2, orow_v, sem)
        pltpu.make_async_copy(deg_hbm.at[cid, pl.ds(0, 16)], orow2_v, sem2).wait()
        scat(j + 1, orow2_v)
        return carry

    lax.fori_loop(0, (NG - 1) // 2, dbody, 0)
    pltpu.make_async_copy(deg_hbm.at[cid, pl.ds(0, 16)], orow_v, sem).wait()
    scat(NG - 1, orow_v)
    plsc.subcore_barrier()
    sl = pl.ds(sid * D_ROWS_TILE, D_ROWS_TILE)
    pltpu.sync_copy(deg_sp.at[sl], st_v)
    pltpu.sync_copy(st_v, deg_hbm.at[cid, sl])


_sc_deg = pl.kernel(
    _sc_deg_body,
    out_type=jax.ShapeDtypeStruct((NC, ND8, 128), jnp.float32),
    mesh=_MESH,
    scratch_types=[
        pltpu.VMEM((E_TILE,), jnp.int32),
        pltpu.VMEM((16, 128), jnp.float32),
        pltpu.VMEM((16, 128), jnp.float32),
        pltpu.VMEM((D_ROWS_TILE, 128), jnp.float32),
        pltpu.VMEM_SHARED((ND8, 128), jnp.float32),
        pltpu.VMEM_SHARED((8, 128), jnp.float32),
        pltpu.SemaphoreType.DMA,
        pltpu.SemaphoreType.DMA,
    ],
)


def _sc_agg_body(src_hbm, dst_hbm, y_hbm, acc_hbm,
                 src_v, dst_v, rows_v, rows2_v, ca_v, acc_sp, sem, sem2):
    cid = lax.axis_index("c")
    sid = lax.axis_index("s")
    g = sid * NC + cid
    zero16 = jnp.zeros((16,), jnp.float32)

    def zrow(i, carry):
        for k in range(8):
            ca_v[i, pl.ds(k * 16, 16)] = zero16
        return carry

    lax.fori_loop(0, CK, zrow, 0)
    a0 = sid * A_ROWS_TILE
    pltpu.sync_copy(ca_v, acc_sp.at[pl.ds(a0, CK)])
    pltpu.sync_copy(ca_v, acc_sp.at[pl.ds(a0 + CK, CK)])
    pltpu.sync_copy(ca_v.at[pl.ds(0, 64)], acc_sp.at[pl.ds(a0 + 2 * CK, 64)])
    pltpu.sync_copy(src_hbm.at[pl.ds(g * E_TILE, E_TILE)], src_v)
    pltpu.sync_copy(dst_hbm.at[pl.ds(g * E_TILE, E_TILE)], dst_v)
    plsc.subcore_barrier()

    one = jnp.full((16,), 1, jnp.int32)
    npad = jnp.full((16,), NPAD, jnp.int32)
    NG = E_TILE // 16

    def fire(j, buf, s):
        s16 = src_v[pl.ds(j * 16, 16)]
        d16 = dst_v[pl.ds(j * 16, 16)]
        pltpu.async_copy(y_hbm.at[s16 + (d16 & one) * npad], buf, s)

    def scat(j, buf):
        d16 = dst_v[pl.ds(j * 16, 16)]
        pltpu.sync_copy(buf, acc_sp.at[lax.shift_right_logical(d16, one)],
                        add=True)

    fire(0, rows_v, sem)

    def abody(i, carry):
        j = 2 * i
        fire(j + 1, rows2_v, sem2)
        pltpu.make_async_copy(y_hbm.at[pl.ds(0, 16)], rows_v, sem).wait()
        scat(j, rows_v)
        fire(j + 2, rows_v, sem)
        pltpu.make_async_copy(y_hbm.at[pl.ds(0, 16)], rows2_v, sem2).wait()
        scat(j + 1, rows2_v)
        return carry

    lax.fori_loop(0, (NG - 1) // 2, abody, 0)
    pltpu.make_async_copy(y_hbm.at[pl.ds(0, 16)], rows_v, sem).wait()
    scat(NG - 1, rows_v)
    plsc.subcore_barrier()
    for c in range(2):
        r0 = a0 + c * CK
        pltpu.sync_copy(acc_sp.at[pl.ds(r0, CK)], ca_v)
        pltpu.sync_copy(ca_v, acc_hbm.at[cid, pl.ds(r0, CK)])
    r0 = a0 + 2 * CK
    pltpu.sync_copy(acc_sp.at[pl.ds(r0, 64)], ca_v.at[pl.ds(0, 64)])
    pltpu.sync_copy(ca_v.at[pl.ds(0, 64)], acc_hbm.at[cid, pl.ds(r0, 64)])


_sc_agg = pl.kernel(
    _sc_agg_body,
    out_type=jax.ShapeDtypeStruct((NC, NH2, 128), jnp.float32),
    mesh=_MESH,
    scratch_types=[
        pltpu.VMEM((E_TILE,), jnp.int32),
        pltpu.VMEM((E_TILE,), jnp.int32),
        pltpu.VMEM((16, 128), jnp.float32),
        pltpu.VMEM((16, 128), jnp.float32),
        pltpu.VMEM((CK, 128), jnp.float32),
        pltpu.VMEM_SHARED((NH2, 128), jnp.float32),
        pltpu.SemaphoreType.DMA,
        pltpu.SemaphoreType.DMA,
    ],
)


RB = 2000
NB = N // RB


def _tc_mid_body(x_ref, w_ref, degp_ref, yd_ref, y_ref):
    half = pl.program_id(0) // NB
    xw = jnp.dot(x_ref[...], w_ref[...], preferred_element_type=jnp.float32)
    deg = degp_ref[0, :, 0:1] + degp_ref[1, :, 0:1] + 1.0
    v = xw * lax.rsqrt(deg)
    y_ref[...] = v
    z = jnp.zeros((RB, H), jnp.float32)

    @pl.when(half == 0)
    def _():
        yd_ref[...] = jnp.concatenate([v, z], axis=1)[None]

    @pl.when(half == 1)
    def _():
        yd_ref[...] = jnp.concatenate([z, v], axis=1)[None]


def _tc_mid(x, w, degp):
    return pl.pallas_call(
        _tc_mid_body,
        grid=(2 * NB,),
        in_specs=[
            pl.BlockSpec((RB, D_IN), lambda j: (j % NB, 0)),
            pl.BlockSpec((D_IN, H), lambda j: (0, 0)),
            pl.BlockSpec((NC, RB, 16), lambda j: (0, j % NB, 0)),
        ],
        out_specs=[
            pl.BlockSpec((1, RB, 128), lambda j: (j // NB, j % NB, 0)),
            pl.BlockSpec((RB, H), lambda j: (j % NB, 0)),
        ],
        out_shape=[
            jax.ShapeDtypeStruct((2, NPAD, 128), jnp.float32),
            jax.ShapeDtypeStruct((N, H), jnp.float32),
        ],
    )(x, w, degp)


def _tc_final_body(acc_ref, degp_ref, y_ref, b_ref,
                   wir_ref, wiz_ref, win_ref, whr_ref, whz_ref, whn_ref,
                   bir_ref, biz_ref, bin_ref, bhr_ref, bhz_ref, bhn_ref,
                   h_ref, out_ref):
    deg = degp_ref[0, :, 0:1] + degp_ref[1, :, 0:1] + 1.0
    dinv = lax.rsqrt(deg)
    y = y_ref[...]
    g = (acc_ref[0] + acc_ref[1] + y) * dinv + b_ref[...]
    h = h_ref[...]
    dot = functools.partial(jnp.dot, preferred_element_type=jnp.float32)
    r = jax.nn.sigmoid(dot(g, wir_ref[...]) + bir_ref[...]
                       + dot(h, whr_ref[...]) + bhr_ref[...])
    z = jax.nn.sigmoid(dot(g, wiz_ref[...]) + biz_ref[...]
                       + dot(h, whz_ref[...]) + bhz_ref[...])
    n = jnp.tanh(dot(g, win_ref[...]) + bin_ref[...]
                 + r * (dot(h, whn_ref[...]) + bhn_ref[...]))
    out_ref[...] = (1.0 - z) * n + z * h


def _tc_final(acc, degp, y, b2, wmats, bvecs, h_mem):
    full = lambda shape: pl.BlockSpec(shape, lambda i: tuple(0 for _ in shape))
    return pl.pallas_call(
        _tc_final_body,
        grid=(NB,),
        in_specs=[
            pl.BlockSpec((NC, RB, H), lambda i: (0, i, 0)),
            pl.BlockSpec((NC, RB, 16), lambda i: (0, i, 0)),
            pl.BlockSpec((RB, H), lambda i: (i, 0)),
            full((1, H)),
            *[full((H, H)) for _ in range(6)],
            *[full((1, H)) for _ in range(6)],
            pl.BlockSpec((RB, H), lambda i: (i, 0)),
        ],
        out_specs=pl.BlockSpec((RB, H), lambda i: (i, 0)),
        out_shape=jax.ShapeDtypeStruct((N, H), jnp.float32),
    )(acc, degp, y, b2, *wmats, *bvecs, h_mem)


def kernel(x, edge_index, W, b, W_ih, W_hh, b_ih, b_hh, h_mem):
    ei = edge_index.astype(jnp.int32)
    src = ei[0].reshape(E)
    dst = ei[1].reshape(E)

    ones8 = jnp.repeat(jnp.eye(8, dtype=jnp.float32), 16, axis=1)

    degp8 = _sc_deg(dst, ones8)
    degp = degp8.reshape(NC, NPAD, 16)
    yd, y = _tc_mid(x, W, degp)
    acc = _sc_agg(src, dst, yd.reshape(2 * NPAD, 128))
    acc = acc.reshape(NC, NPAD, H)

    wm = [W_ih[0:H].T, W_ih[H:2 * H].T, W_ih[2 * H:3 * H].T,
          W_hh[0:H].T, W_hh[H:2 * H].T, W_hh[2 * H:3 * H].T]
    bv = [b_ih[0:H].reshape(1, H), b_ih[H:2 * H].reshape(1, H),
          b_ih[2 * H:3 * H].reshape(1, H),
          b_hh[0:H].reshape(1, H), b_hh[H:2 * H].reshape(1, H),
          b_hh[2 * H:3 * H].reshape(1, H)]
    return _tc_final(acc, degp, y, b.reshape(1, H), wm, bv, h_mem)


# speedup vs baseline: 18.7422x; 1.1995x over previous
import functools

import jax
import jax.numpy as jnp
from jax import lax
from jax.experimental import pallas as pl
from jax.experimental.pallas import tpu as pltpu
from jax.experimental.pallas import tpu_sc as plsc

N = 10000
E = 320000
D_IN = 128
H = 64

NC = 2
NS = 16
NW = NC * NS

NPAD = 10240
ND8 = NPAD // 8
NH2 = NPAD // 2
D_ROWS_TILE = ND8 // NS
A_ROWS_TILE = NH2 // NS
E_TILE = E // NW
CK = 128

_MESH = plsc.VectorSubcoreMesh(
    core_axis_name="c", subcore_axis_name="s", num_cores=NC, num_subcores=NS
)


def _sc_deg_body(dst_hbm, ones8_hbm, deg_hbm, dst_v, orow_v, orow2_v,
                 orow3_v, orow4_v, st_v, deg_sp, ones_sp, gsems, ssems):
    cid = lax.axis_index("c")
    sid = lax.axis_index("s")
    g = sid * NC + cid
    zero16 = jnp.zeros((16,), jnp.float32)

    def zrow(i, carry):
        for k in range(8):
            st_v[i, pl.ds(k * 16, 16)] = zero16
        return carry

    lax.fori_loop(0, D_ROWS_TILE, zrow, 0)
    pltpu.sync_copy(st_v, deg_sp.at[pl.ds(sid * D_ROWS_TILE, D_ROWS_TILE)])

    @pl.when(sid == 0)
    def _():
        pltpu.sync_copy(ones8_hbm, st_v.at[pl.ds(0, 8)])
        pltpu.sync_copy(st_v.at[pl.ds(0, 8)], ones_sp)

    pltpu.sync_copy(dst_hbm.at[pl.ds(g * E_TILE, E_TILE)], dst_v)
    plsc.subcore_barrier()

    seven = jnp.full((16,), 7, jnp.int32)
    three = jnp.full((16,), 3, jnp.int32)
    NG = E_TILE // 16
    NQ = NG // 4
    bufs = [orow_v, orow2_v, orow3_v, orow4_v]

    def fire(j, buf, s):
        d16 = dst_v[pl.ds(j * 16, 16)]
        pltpu.async_copy(ones_sp.at[d16 & seven], buf, s)

    def gwait(buf, s):
        pltpu.make_async_copy(deg_hbm.at[cid, pl.ds(0, 16)], buf, s).wait()

    def sfire(j, buf, s):
        d16 = dst_v[pl.ds(j * 16, 16)]
        pltpu.async_copy(buf, deg_sp.at[lax.shift_right_logical(d16, three)],
                         s, add=True)

    def swait(buf, s):
        pltpu.make_async_copy(buf, deg_sp.at[pl.ds(0, 16)], s).wait()

    for k in range(4):
        fire(k, bufs[k], gsems[k])

    def dbody(i, carry):
        for k in range(4):
            gwait(bufs[k], gsems[k])
            sfire(4 * i + k, bufs[k], ssems[k])
        for k in range(4):
            swait(bufs[k], ssems[k])
            fire(4 * (i + 1) + k, bufs[k], gsems[k])
        return carry

    lax.fori_loop(0, NQ - 1, dbody, 0)
    for k in range(4):
        gwait(bufs[k], gsems[k])
        sfire(4 * (NQ - 1) + k, bufs[k], ssems[k])
    for k in range(4):
        swait(bufs[k], ssems[k])
    fire(NG - 1, bufs[0], gsems[0])
    gwait(bufs[0], gsems[0])
    sfire(NG - 1, bufs[0], ssems[0])
    swait(bufs[0], ssems[0])
    plsc.subcore_barrier()
    sl = pl.ds(sid * D_ROWS_TILE, D_ROWS_TILE)
    pltpu.sync_copy(deg_sp.at[sl], st_v)
    pltpu.sync_copy(st_v, deg_hbm.at[cid, sl])


_sc_deg = pl.kernel(
    _sc_deg_body,
    out_type=jax.ShapeDtypeStruct((NC, ND8, 128), jnp.float32),
    mesh=_MESH,
    scratch_types=[
        pltpu.VMEM((E_TILE,), jnp.int32),
        pltpu.VMEM((16, 128), jnp.float32),
        pltpu.VMEM((16, 128), jnp.float32),
        pltpu.VMEM((16, 128), jnp.float32),
        pltpu.VMEM((16, 128), jnp.float32),
        pltpu.VMEM((D_ROWS_TILE, 128), jnp.float32),
        pltpu.VMEM_SHARED((ND8, 128), jnp.float32),
        pltpu.VMEM_SHARED((8, 128), jnp.float32),
        [pltpu.SemaphoreType.DMA] * 4,
        [pltpu.SemaphoreType.DMA] * 4,
    ],
)


def _sc_agg_body(src_hbm, dst_hbm, y_hbm, acc_hbm,
                 src_v, dst_v, rows_v, rows2_v, rows3_v, rows4_v, ca_v,
                 acc_sp, gsems, ssems):
    cid = lax.axis_index("c")
    sid = lax.axis_index("s")
    g = sid * NC + cid
    zero16 = jnp.zeros((16,), jnp.float32)

    def zrow(i, carry):
        for k in range(8):
            ca_v[i, pl.ds(k * 16, 16)] = zero16
        return carry

    lax.fori_loop(0, CK, zrow, 0)
    a0 = sid * A_ROWS_TILE
    pltpu.sync_copy(ca_v, acc_sp.at[pl.ds(a0, CK)])
    pltpu.sync_copy(ca_v, acc_sp.at[pl.ds(a0 + CK, CK)])
    pltpu.sync_copy(ca_v.at[pl.ds(0, 64)], acc_sp.at[pl.ds(a0 + 2 * CK, 64)])
    pltpu.sync_copy(src_hbm.at[pl.ds(g * E_TILE, E_TILE)], src_v)
    pltpu.sync_copy(dst_hbm.at[pl.ds(g * E_TILE, E_TILE)], dst_v)
    plsc.subcore_barrier()

    one = jnp.full((16,), 1, jnp.int32)
    npad = jnp.full((16,), NPAD, jnp.int32)
    NG = E_TILE // 16
    NQ = NG // 4
    bufs = [rows_v, rows2_v, rows3_v, rows4_v]

    def fire(j, buf, s):
        s16 = src_v[pl.ds(j * 16, 16)]
        d16 = dst_v[pl.ds(j * 16, 16)]
        pltpu.async_copy(y_hbm.at[s16 + (d16 & one) * npad], buf, s)

    def gwait(buf, s):
        pltpu.make_async_copy(y_hbm.at[pl.ds(0, 16)], buf, s).wait()

    def sfire(j, buf, s):
        d16 = dst_v[pl.ds(j * 16, 16)]
        pltpu.async_copy(buf, acc_sp.at[lax.shift_right_logical(d16, one)],
                         s, add=True)

    def swait(buf, s):
        pltpu.make_async_copy(buf, acc_sp.at[pl.ds(0, 16)], s).wait()

    for k in range(4):
        fire(k, bufs[k], gsems[k])

    def abody(i, carry):
        for k in range(4):
            gwait(bufs[k], gsems[k])
            sfire(4 * i + k, bufs[k], ssems[k])
        for k in range(4):
            swait(bufs[k], ssems[k])
            fire(4 * (i + 1) + k, bufs[k], gsems[k])
        return carry

    lax.fori_loop(0, NQ - 1, abody, 0)
    for k in range(4):
        gwait(bufs[k], gsems[k])
        sfire(4 * (NQ - 1) + k, bufs[k], ssems[k])
    for k in range(4):
        swait(bufs[k], ssems[k])
    fire(NG - 1, bufs[0], gsems[0])
    gwait(bufs[0], gsems[0])
    sfire(NG - 1, bufs[0], ssems[0])
    swait(bufs[0], ssems[0])
    plsc.subcore_barrier()
    for c in range(2):
        r0 = a0 + c * CK
        pltpu.sync_copy(acc_sp.at[pl.ds(r0, CK)], ca_v)
        pltpu.sync_copy(ca_v, acc_hbm.at[cid, pl.ds(r0, CK)])
    r0 = a0 + 2 * CK
    pltpu.sync_copy(acc_sp.at[pl.ds(r0, 64)], ca_v.at[pl.ds(0, 64)])
    pltpu.sync_copy(ca_v.at[pl.ds(0, 64)], acc_hbm.at[cid, pl.ds(r0, 64)])


_sc_agg = pl.kernel(
    _sc_agg_body,
    out_type=jax.ShapeDtypeStruct((NC, NH2, 128), jnp.float32),
    mesh=_MESH,
    scratch_types=[
        pltpu.VMEM((E_TILE,), jnp.int32),
        pltpu.VMEM((E_TILE,), jnp.int32),
        pltpu.VMEM((16, 128), jnp.float32),
        pltpu.VMEM((16, 128), jnp.float32),
        pltpu.VMEM((16, 128), jnp.float32),
        pltpu.VMEM((16, 128), jnp.float32),
        pltpu.VMEM((CK, 128), jnp.float32),
        pltpu.VMEM_SHARED((NH2, 128), jnp.float32),
        [pltpu.SemaphoreType.DMA] * 4,
        [pltpu.SemaphoreType.DMA] * 4,
    ],
)


RB = 2000
NB = N // RB


def _tc_mid_body(x_ref, w_ref, degp_ref, yd_ref, y_ref):
    half = pl.program_id(0) // NB
    xw = jnp.dot(x_ref[...], w_ref[...], preferred_element_type=jnp.float32)
    deg = degp_ref[0, :, 0:1] + degp_ref[1, :, 0:1] + 1.0
    v = xw * lax.rsqrt(deg)
    y_ref[...] = v
    z = jnp.zeros((RB, H), jnp.float32)

    @pl.when(half == 0)
    def _():
        yd_ref[...] = jnp.concatenate([v, z], axis=1)[None]

    @pl.when(half == 1)
    def _():
        yd_ref[...] = jnp.concatenate([z, v], axis=1)[None]


def _tc_mid(x, w, degp):
    return pl.pallas_call(
        _tc_mid_body,
        grid=(2 * NB,),
        in_specs=[
            pl.BlockSpec((RB, D_IN), lambda j: (j % NB, 0)),
            pl.BlockSpec((D_IN, H), lambda j: (0, 0)),
            pl.BlockSpec((NC, RB, 16), lambda j: (0, j % NB, 0)),
        ],
        out_specs=[
            pl.BlockSpec((1, RB, 128), lambda j: (j // NB, j % NB, 0)),
            pl.BlockSpec((RB, H), lambda j: (j % NB, 0)),
        ],
        out_shape=[
            jax.ShapeDtypeStruct((2, NPAD, 128), jnp.float32),
            jax.ShapeDtypeStruct((N, H), jnp.float32),
        ],
    )(x, w, degp)


def _tc_final_body(acc_ref, degp_ref, y_ref, b_ref,
                   wir_ref, wiz_ref, win_ref, whr_ref, whz_ref, whn_ref,
                   bir_ref, biz_ref, bin_ref, bhr_ref, bhz_ref, bhn_ref,
                   h_ref, out_ref):
    deg = degp_ref[0, :, 0:1] + degp_ref[1, :, 0:1] + 1.0
    dinv = lax.rsqrt(deg)
    y = y_ref[...]
    g = (acc_ref[0] + acc_ref[1] + y) * dinv + b_ref[...]
    h = h_ref[...]
    dot = functools.partial(jnp.dot, preferred_element_type=jnp.float32)
    r = jax.nn.sigmoid(dot(g, wir_ref[...]) + bir_ref[...]
                       + dot(h, whr_ref[...]) + bhr_ref[...])
    z = jax.nn.sigmoid(dot(g, wiz_ref[...]) + biz_ref[...]
                       + dot(h, whz_ref[...]) + bhz_ref[...])
    n = jnp.tanh(dot(g, win_ref[...]) + bin_ref[...]
                 + r * (dot(h, whn_ref[...]) + bhn_ref[...]))
    out_ref[...] = (1.0 - z) * n + z * h


def _tc_final(acc, degp, y, b2, wmats, bvecs, h_mem):
    full = lambda shape: pl.BlockSpec(shape, lambda i: tuple(0 for _ in shape))
    return pl.pallas_call(
        _tc_final_body,
        grid=(NB,),
        in_specs=[
            pl.BlockSpec((NC, RB, H), lambda i: (0, i, 0)),
            pl.BlockSpec((NC, RB, 16), lambda i: (0, i, 0)),
            pl.BlockSpec((RB, H), lambda i: (i, 0)),
            full((1, H)),
            *[full((H, H)) for _ in range(6)],
            *[full((1, H)) for _ in range(6)],
            pl.BlockSpec((RB, H), lambda i: (i, 0)),
        ],
        out_specs=pl.BlockSpec((RB, H), lambda i: (i, 0)),
        out_shape=jax.ShapeDtypeStruct((N, H), jnp.float32),
    )(acc, degp, y, b2, *wmats, *bvecs, h_mem)


def kernel(x, edge_index, W, b, W_ih, W_hh, b_ih, b_hh, h_mem):
    ei = edge_index.astype(jnp.int32)
    src = ei[0].reshape(E)
    dst = ei[1].reshape(E)

    ones8 = jnp.repeat(jnp.eye(8, dtype=jnp.float32), 16, axis=1)

    degp8 = _sc_deg(dst, ones8)
    degp = degp8.reshape(NC, NPAD, 16)
    yd, y = _tc_mid(x, W, degp)
    acc = _sc_agg(src, dst, yd.reshape(2 * NPAD, 128))
    acc = acc.reshape(NC, NPAD, H)

    wm = [W_ih[0:H].T, W_ih[H:2 * H].T, W_ih[2 * H:3 * H].T,
          W_hh[0:H].T, W_hh[H:2 * H].T, W_hh[2 * H:3 * H].T]
    bv = [b_ih[0:H].reshape(1, H), b_ih[H:2 * H].reshape(1, H),
          b_ih[2 * H:3 * H].reshape(1, H),
          b_hh[0:H].reshape(1, H), b_hh[H:2 * H].reshape(1, H),
          b_hh[2 * H:3 * H].reshape(1, H)]
    return _tc_final(acc, degp, y, b.reshape(1, H), wm, bv, h_mem)
